# baseline (speedup 1.0000x reference)
import functools

import jax
import jax.numpy as jnp
from jax import lax
from jax.experimental import pallas as pl
from jax.experimental.pallas import tpu as pltpu
from jax.experimental.pallas import tpu_sc as plsc

_NC = 2
_NS = 16
_NW = _NC * _NS
_G = 128


def _build(num_granules, dim):
    mesh = plsc.VectorSubcoreMesh(core_axis_name="c", subcore_axis_name="s")

    @functools.partial(
        pl.kernel,
        mesh=mesh,
        compiler_params=pltpu.CompilerParams(use_tc_tiling_on_sc=False),
        out_type=jax.ShapeDtypeStruct((_NW, num_granules, _G, dim), jnp.float32),
        scratch_types=[
            pltpu.VMEM((num_granules, _G), jnp.int32),
            pltpu.VMEM((_G, dim), jnp.float32),
            pltpu.SemaphoreType.DMA,
        ],
    )
    def body(ids_hbm, table_hbm, out_hbm, idx_v, rows_v, sem):
        wid = lax.axis_index("s") * _NC + lax.axis_index("c")
        pltpu.sync_copy(ids_hbm.at[wid], idx_v)

        def step(g, carry):
            pltpu.async_copy(table_hbm.at[idx_v.at[g]], rows_v, sem).wait()
            pltpu.sync_copy(rows_v, out_hbm.at[wid, g])
            return carry

        lax.fori_loop(0, num_granules, step, 0)

    return body


def kernel(token_ids, weight):
    batch, seq = token_ids.shape
    dim = weight.shape[1]
    ids = token_ids.reshape(-1).astype(jnp.int32)
    num_granules = ids.shape[0] // (_NW * _G)
    out = _build(num_granules, dim)(ids.reshape(_NW, num_granules, _G), weight)
    return out.reshape(batch, seq, dim)
